# baseline (speedup 1.0000x reference)
import functools

import jax
import jax.numpy as jnp
from jax import lax
from jax.experimental import pallas as pl
from jax.experimental.pallas import tpu as pltpu
from jax.experimental.pallas import tpu_sc as plsc

_LANES = 128
_PB = 8


def _sc_gather(table, xpad, Bt, S, D, SP):
    V, _ = table.shape
    info = plsc.get_sparse_core_info()
    NC, NS = info.num_cores, info.num_subcores
    NW = NC * NS
    assert Bt % NW == 0
    b_per_w = Bt // NW
    assert b_per_w % _PB == 0
    n_super = b_per_w // _PB
    assert n_super % 2 == 0

    mesh = plsc.VectorSubcoreMesh(core_axis_name="c", subcore_axis_name="s")

    @functools.partial(
        pl.kernel,
        mesh=mesh,
        compiler_params=pltpu.CompilerParams(use_tc_tiling_on_sc=False),
        out_type=jax.ShapeDtypeStruct((Bt, SP, _LANES), jnp.float32),
        scratch_types=[
            pltpu.VMEM((b_per_w, _LANES), jnp.int32),
            pltpu.VMEM((_PB, SP, D), jnp.float32),
            pltpu.VMEM((_PB, SP, D), jnp.float32),
            pltpu.SemaphoreType.DMA,
            pltpu.SemaphoreType.DMA,
            pltpu.SemaphoreType.DMA,
            pltpu.SemaphoreType.DMA,
        ],
    )
    def k(table_hbm, x_hbm, out_hbm, idx_v, rows0, rows1, g0, g1, o0, o1):
        wid = lax.axis_index("s") * NC + lax.axis_index("c")
        wbase = wid * b_per_w
        pltpu.sync_copy(x_hbm.at[pl.ds(wbase, b_per_w)], idx_v)
        bufs = (rows0, rows1)
        gsems = (g0, g1)
        osems = (o0, o1)

        def superchunk(sc, half, first):
            buf, gsem, osem = bufs[half], gsems[half], osems[half]
            @pl.when(jnp.logical_not(first))
            def _():
                pltpu.make_async_copy(
                    buf,
                    out_hbm.at[pl.ds(wbase, _PB), pl.ds(0, SP), pl.ds(0, D)],
                    osem,
                ).wait()
            descs = []
            for bb in range(_PB):
                d = pltpu.async_copy(
                    table_hbm.at[idx_v.at[sc * _PB + bb, pl.ds(0, SP)]],
                    buf.at[bb],
                    gsem,
                )
                descs.append(d)
            for d in descs:
                d.wait()
            pltpu.async_copy(
                buf,
                out_hbm.at[
                    pl.ds(wbase + sc * _PB, _PB), pl.ds(0, SP), pl.ds(0, D)
                ],
                osem,
            )

        def body(i2, carry):
            superchunk(2 * i2, 0, i2 == 0)
            superchunk(2 * i2 + 1, 1, i2 == 0)
            return carry

        lax.fori_loop(0, n_super // 2, body, 0)
        for half in range(2):
            pltpu.make_async_copy(
                bufs[half],
                out_hbm.at[pl.ds(wbase, _PB), pl.ds(0, SP), pl.ds(0, D)],
                osems[half],
            ).wait()

    return k(table, xpad)


_SQRT_HALF = 0.7071067811865476


def _make_adapter_body(BB, S, D, SP):
    def body(x_ref, w_ref, b_ref, o_ref):
        v = x_ref[...].reshape(BB * SP, _LANES)
        col = lax.broadcasted_iota(jnp.int32, (BB * SP, _LANES), 1)
        v = jnp.where(col < D, v, 0.0)
        h = jnp.dot(v, w_ref[...], preferred_element_type=jnp.float32)
        h = h + b_ref[...]
        g = h * 0.5 * (1.0 + lax.erf(h * _SQRT_HALF))
        o_ref[...] = g.reshape(BB, SP, _LANES)[:, :S, :D]

    return body


def _tc_adapter(Gp, W128, b128, Bt, S, D, SP):
    BB = 128
    assert Bt % BB == 0
    return pl.pallas_call(
        _make_adapter_body(BB, S, D, SP),
        grid=(Bt // BB,),
        in_specs=[
            pl.BlockSpec((BB, SP, _LANES), lambda i: (i, 0, 0)),
            pl.BlockSpec((_LANES, _LANES), lambda i: (0, 0)),
            pl.BlockSpec((1, _LANES), lambda i: (0, 0)),
        ],
        out_specs=pl.BlockSpec((BB, S, D), lambda i: (i, 0, 0)),
        out_shape=jax.ShapeDtypeStruct((Bt, S, D), jnp.float32),
    )(Gp, W128, b128)


def kernel(x, table, W, b):
    Bt, S = x.shape
    V, D = table.shape
    SP = ((S + 7) // 8) * 8
    xpad = jnp.pad(x.astype(jnp.int32), ((0, 0), (0, _LANES - S)))
    Gp = _sc_gather(table, xpad, Bt, S, D, SP)
    W128 = jnp.zeros((_LANES, _LANES), jnp.float32).at[:D, :D].set(W.T)
    b128 = jnp.zeros((1, _LANES), jnp.float32).at[0, :D].set(b)
    return _tc_adapter(Gp, W128, b128, Bt, S, D, SP)
